# Initial kernel scaffold; baseline (speedup 1.0000x reference)
import functools

import jax
import jax.numpy as jnp
from jax import lax
from jax.experimental import pallas as pl
from jax.experimental.pallas import tpu as pltpu
from jax.experimental.pallas import tpu_sc as plsc

_N = 10000
_E = 320000
_D = 128
_L = 5

_NC = 2
_NS = 16
_NW = _NC * _NS
_EPW = _E // _NW
_CHUNK = 80
_NCHUNK = _EPW // _CHUNK
_RPS = _N // _NS
_ZROWS = 125


def _sc_agg_body(h_hbm, src_hbm, dst_hbm, ea_hbm, out_hbm,
                 agg_sh, idx_s, idx_d, rows, ea, zbuf, sem):
    c = lax.axis_index("c")
    s = lax.axis_index("s")
    w = c * _NS + s

    zero16 = jnp.zeros((16,), jnp.float32)

    def zrow(r, carry):
        for j in range(_D // 16):
            zbuf[r, pl.ds(j * 16, 16)] = zero16
        return carry

    lax.fori_loop(0, _ZROWS, zrow, 0)
    for k in range(_RPS // _ZROWS):
        pltpu.sync_copy(zbuf, agg_sh.at[pl.ds(s * _RPS + k * _ZROWS, _ZROWS), :])
    plsc.subcore_barrier()

    base = w * _EPW

    def chunk(i, carry):
        off = base + i * _CHUNK
        pltpu.sync_copy(src_hbm.at[pl.ds(off, _CHUNK)], idx_s)
        pltpu.sync_copy(dst_hbm.at[pl.ds(off, _CHUNK)], idx_d)
        pltpu.sync_copy(ea_hbm.at[pl.ds(off, _CHUNK), :], ea)
        pltpu.async_copy(h_hbm.at[idx_s], rows, sem).wait()

        def rbody(r, cc):
            for j in range(_D // 16):
                sl = pl.ds(j * 16, 16)
                rows[r, sl] = jnp.maximum(rows[r, sl] + ea[r, sl], 0.0)
            return cc

        lax.fori_loop(0, _CHUNK, rbody, 0)
        pltpu.sync_copy(rows, agg_sh.at[idx_d], add=True)
        return carry

    lax.fori_loop(0, _NCHUNK, chunk, 0)

    plsc.subcore_barrier()
    pltpu.sync_copy(agg_sh.at[pl.ds(s * _RPS, _RPS), :],
                    out_hbm.at[c, pl.ds(s * _RPS, _RPS), :])


_sc_agg = pl.kernel(
    _sc_agg_body,
    out_type=jax.ShapeDtypeStruct((_NC, _N, _D), jnp.float32),
    mesh=plsc.VectorSubcoreMesh(core_axis_name="c", subcore_axis_name="s",
                                num_cores=_NC, num_subcores=_NS),
    scratch_types=[
        pltpu.VMEM_SHARED((_N, _D), jnp.float32),
        pltpu.VMEM((_CHUNK,), jnp.int32),
        pltpu.VMEM((_CHUNK,), jnp.int32),
        pltpu.VMEM((_CHUNK, _D), jnp.float32),
        pltpu.VMEM((_CHUNK, _D), jnp.float32),
        pltpu.VMEM((_ZROWS, _D), jnp.float32),
        pltpu.SemaphoreType.DMA,
    ],
)


def _mlp_body(scale_ref, h_ref, agg_ref, w1_ref, b1_ref, w2_ref, b2_ref,
              out_ref):
    t = scale_ref[0] * h_ref[...] + agg_ref[0] + agg_ref[1]
    t = jnp.dot(t, w1_ref[...], preferred_element_type=jnp.float32)
    t = jnp.maximum(t + b1_ref[...], 0.0)
    t = jnp.dot(t, w2_ref[...], preferred_element_type=jnp.float32)
    out_ref[...] = jnp.maximum(t + b2_ref[...], 0.0)


_BN = 1000


def _tc_mlp(h, agg, w1, b1, w2, b2, eps_l):
    scale = (1.0 + eps_l).reshape(1)
    return pl.pallas_call(
        _mlp_body,
        grid=(_N // _BN,),
        in_specs=[
            pl.BlockSpec(memory_space=pltpu.SMEM),
            pl.BlockSpec((_BN, _D), lambda i: (i, 0)),
            pl.BlockSpec((_NC, _BN, _D), lambda i: (0, i, 0)),
            pl.BlockSpec((_D, _D), lambda i: (0, 0)),
            pl.BlockSpec((1, _D), lambda i: (0, 0)),
            pl.BlockSpec((_D, _D), lambda i: (0, 0)),
            pl.BlockSpec((1, _D), lambda i: (0, 0)),
        ],
        out_specs=pl.BlockSpec((_BN, _D), lambda i: (i, 0)),
        out_shape=jax.ShapeDtypeStruct((_N, _D), jnp.float32),
    )(scale, h, agg, w1, b1.reshape(1, _D), w2, b2.reshape(1, _D))


def kernel(x, edge_index, edge_attr, W1, b1, W2, b2, eps):
    src = edge_index[0]
    dst = edge_index[1]
    h = x
    for l in range(_L):
        agg = _sc_agg(h, src, dst, edge_attr)
        h = _tc_mlp(h, agg, W1[l], b1[l], W2[l], b2[l], eps[l])
    return h


# speedup vs baseline: 2.9589x; 2.9589x over previous
import functools

import jax
import jax.numpy as jnp
from jax import lax
from jax.experimental import pallas as pl
from jax.experimental.pallas import tpu as pltpu
from jax.experimental.pallas import tpu_sc as plsc

_N = 10000
_E = 320000
_D = 128
_L = 5

_NC = 2
_NS = 16
_NW = _NC * _NS
_EPW = _E // _NW
_CHUNK = 80
_NCHUNK = _EPW // _CHUNK
_ZROWS = 80
_ZCHUNKS = 8


def _sc_agg_body(h_hbm, src_hbm, dst_hbm, ea_hbm, out_hbm,
                 agg_sh, idx_s, idx_d, rows, ea, zbuf, sem):
    c = lax.axis_index("c")
    s = lax.axis_index("s")
    w = c * _NS + s

    zero16 = jnp.zeros((16,), jnp.float32)

    def zrow(r, carry):
        for j in range(_D // 16):
            zbuf[r, pl.ds(j * 16, 16)] = zero16
        return carry

    lax.fori_loop(0, _ZROWS, zrow, 0)
    for k in range(_ZCHUNKS):
        g = s * (_ZROWS * _ZCHUNKS) + k * _ZROWS

        @pl.when(g < _N)
        def _():
            pltpu.sync_copy(zbuf, agg_sh.at[pl.ds(g, _ZROWS), :])

    plsc.subcore_barrier()

    base = w * _EPW

    def chunk(i, carry):
        off = base + i * _CHUNK
        pltpu.sync_copy(src_hbm.at[pl.ds(off, _CHUNK)], idx_s)
        pltpu.sync_copy(dst_hbm.at[pl.ds(off, _CHUNK)], idx_d)
        pltpu.sync_copy(ea_hbm.at[pl.ds(off, _CHUNK), :], ea)
        pltpu.async_copy(h_hbm.at[idx_s], rows, sem).wait()

        def rbody(r, cc):
            for j in range(_D // 16):
                sl = pl.ds(j * 16, 16)
                rows[r, sl] = jnp.maximum(rows[r, sl] + ea[r, sl], 0.0)
            return cc

        lax.fori_loop(0, _CHUNK, rbody, 0)
        pltpu.sync_copy(rows, agg_sh.at[idx_d], add=True)
        return carry

    lax.fori_loop(0, _NCHUNK, chunk, 0)

    plsc.subcore_barrier()
    for k in range(_ZCHUNKS):
        g = s * (_ZROWS * _ZCHUNKS) + k * _ZROWS

        @pl.when(g < _N)
        def _():
            pltpu.sync_copy(agg_sh.at[pl.ds(g, _ZROWS), :],
                            out_hbm.at[c, pl.ds(g, _ZROWS), :])


_sc_agg = pl.kernel(
    _sc_agg_body,
    out_type=jax.ShapeDtypeStruct((_NC, _N, _D), jnp.float32),
    mesh=plsc.VectorSubcoreMesh(core_axis_name="c", subcore_axis_name="s",
                                num_cores=_NC, num_subcores=_NS),
    scratch_types=[
        pltpu.VMEM_SHARED((_N, _D), jnp.float32),
        pltpu.VMEM((_CHUNK,), jnp.int32),
        pltpu.VMEM((_CHUNK,), jnp.int32),
        pltpu.VMEM((_CHUNK, _D), jnp.float32),
        pltpu.VMEM((_CHUNK, _D), jnp.float32),
        pltpu.VMEM((_ZROWS, _D), jnp.float32),

        pltpu.SemaphoreType.DMA,
    ],
)


def _mlp_body(scale_ref, h_ref, agg_ref, w1_ref, b1_ref, w2_ref, b2_ref,
              out_ref):
    t = scale_ref[0] * h_ref[...] + agg_ref[0] + agg_ref[1]
    t = jnp.dot(t, w1_ref[...], preferred_element_type=jnp.float32)
    t = jnp.maximum(t + b1_ref[...], 0.0)
    t = jnp.dot(t, w2_ref[...], preferred_element_type=jnp.float32)
    out_ref[...] = jnp.maximum(t + b2_ref[...], 0.0)


_BN = 1000


def _tc_mlp(h, agg, w1, b1, w2, b2, eps_l):
    scale = (1.0 + eps_l).reshape(1)
    return pl.pallas_call(
        _mlp_body,
        grid=(_N // _BN,),
        in_specs=[
            pl.BlockSpec(memory_space=pltpu.SMEM),
            pl.BlockSpec((_BN, _D), lambda i: (i, 0)),
            pl.BlockSpec((_NC, _BN, _D), lambda i: (0, i, 0)),
            pl.BlockSpec((_D, _D), lambda i: (0, 0)),
            pl.BlockSpec((1, _D), lambda i: (0, 0)),
            pl.BlockSpec((_D, _D), lambda i: (0, 0)),
            pl.BlockSpec((1, _D), lambda i: (0, 0)),
        ],
        out_specs=pl.BlockSpec((_BN, _D), lambda i: (i, 0)),
        out_shape=jax.ShapeDtypeStruct((_N, _D), jnp.float32),
    )(scale, h, agg, w1, b1.reshape(1, _D), w2, b2.reshape(1, _D))


def kernel(x, edge_index, edge_attr, W1, b1, W2, b2, eps):
    src = edge_index[0]
    dst = edge_index[1]
    h = x
    for l in range(_L):
        agg = _sc_agg(h, src, dst, edge_attr)
        h = _tc_mlp(h, agg, W1[l], b1[l], W2[l], b2[l], eps[l])
    return h


# speedup vs baseline: 7.4026x; 2.5019x over previous
import functools

import jax
import jax.numpy as jnp
from jax import lax
from jax.experimental import pallas as pl
from jax.experimental.pallas import tpu as pltpu
from jax.experimental.pallas import tpu_sc as plsc

_N = 10000
_E = 320000
_D = 128
_L = 5

_NC = 2
_NS = 16
_NW = _NC * _NS
_EPW = _E // _NW
_CHUNK = 40
_NCHUNK = _EPW // _CHUNK
_ZROWS = 40
_ZCHUNKS = 16


_NBUF = 4
_NGRP = _NCHUNK // _NBUF
_NPEEL = _NCHUNK - _NGRP * _NBUF


def _sc_agg_body(h_hbm, src_hbm, dst_hbm, ea_hbm, out_hbm,
                 agg_sh, idx_s, idx_d, rows, ea,
                 sem_pre, sem_g, sem_e, sem_sc):
    c = lax.axis_index("c")
    s = lax.axis_index("s")
    w = c * _NS + s
    base = w * _EPW

    def fire_idx(q, j):
        off = base + j * _CHUNK
        pltpu.async_copy(src_hbm.at[pl.ds(off, _CHUNK)], idx_s.at[q],
                         sem_pre.at[q])
        pltpu.async_copy(dst_hbm.at[pl.ds(off, _CHUNK)], idx_d.at[q],
                         sem_pre.at[q])

    def issue_ge(q, j):
        off = base + j * _CHUNK
        pltpu.make_async_copy(src_hbm.at[pl.ds(off, _CHUNK)], idx_s.at[q],
                              sem_pre.at[q]).wait()
        pltpu.make_async_copy(dst_hbm.at[pl.ds(off, _CHUNK)], idx_d.at[q],
                              sem_pre.at[q]).wait()
        pltpu.async_copy(ea_hbm.at[pl.ds(off, _CHUNK), :], ea.at[q],
                         sem_e.at[q])
        pltpu.async_copy(h_hbm.at[idx_s.at[q]], rows.at[q], sem_g.at[q])

    def wait_sc(q):
        pltpu.make_async_copy(rows.at[q], agg_sh.at[idx_d.at[q]],
                              sem_sc.at[q]).wait()

    zero16 = jnp.zeros((16,), jnp.float32)

    def zrow(r, carry):
        for j in range(_D // 16):
            ea[0, r, pl.ds(j * 16, 16)] = zero16
        return carry

    lax.fori_loop(0, _ZROWS, zrow, 0)
    for k in range(_ZCHUNKS):
        g = s * (_ZROWS * _ZCHUNKS) + k * _ZROWS

        @pl.when(g < _N)
        def _():
            pltpu.sync_copy(ea.at[0], agg_sh.at[pl.ds(g, _ZROWS), :])

    plsc.subcore_barrier()

    def do_chunk(b, j):
        pltpu.make_async_copy(h_hbm.at[idx_s.at[b]], rows.at[b],
                              sem_g.at[b]).wait()
        pltpu.make_async_copy(ea_hbm.at[pl.ds(0, _CHUNK), :], ea.at[b],
                              sem_e.at[b]).wait()

        def rbody(r, cc, b=b):
            for jj in range(_D // 16):
                sl = pl.ds(jj * 16, 16)
                rows[b, r, sl] = jnp.maximum(
                    rows[b, r, sl] + ea[b, r, sl], 0.0)
            return cc

        lax.fori_loop(0, _CHUNK, rbody, 0)
        pltpu.async_copy(rows.at[b], agg_sh.at[idx_d.at[b]],
                         sem_sc.at[b], add=True)

        q3 = (b + 3) % _NBUF
        q2 = (b + 2) % _NBUF

        @pl.when(jnp.logical_and(j >= 1, j <= _NCHUNK - 4))
        def _():
            wait_sc(q3)

        @pl.when(j <= _NCHUNK - 4)
        def _():
            fire_idx(q3, j + 3)

        @pl.when(j <= _NCHUNK - 3)
        def _():
            issue_ge(q2, j + 2)

    fire_idx(0, 0)
    fire_idx(1, 1)
    fire_idx(2, 2)
    issue_ge(0, 0)
    issue_ge(1, 1)

    def group(g, carry):
        for b in range(_NBUF):
            do_chunk(b, g * _NBUF + b)
        return carry

    lax.fori_loop(0, _NGRP, group, 0)
    for t in range(_NPEEL):
        do_chunk(t, jnp.int32(_NGRP * _NBUF + t))

    for q in range(_NBUF):
        wait_sc(q)
    plsc.subcore_barrier()
    for k in range(_ZCHUNKS):
        g = s * (_ZROWS * _ZCHUNKS) + k * _ZROWS

        @pl.when(g < _N)
        def _():
            pltpu.sync_copy(agg_sh.at[pl.ds(g, _ZROWS), :],
                            out_hbm.at[c, pl.ds(g, _ZROWS), :])


_sc_agg = pl.kernel(
    _sc_agg_body,
    out_type=jax.ShapeDtypeStruct((_NC, _N, _D), jnp.float32),
    mesh=plsc.VectorSubcoreMesh(core_axis_name="c", subcore_axis_name="s",
                                num_cores=_NC, num_subcores=_NS),
    scratch_types=[
        pltpu.VMEM_SHARED((_N, _D), jnp.float32),
        pltpu.VMEM((_NBUF, _CHUNK), jnp.int32),
        pltpu.VMEM((_NBUF, _CHUNK), jnp.int32),
        pltpu.VMEM((_NBUF, _CHUNK, _D), jnp.float32),
        pltpu.VMEM((_NBUF, _CHUNK, _D), jnp.float32),
        pltpu.SemaphoreType.DMA((_NBUF,)),
        pltpu.SemaphoreType.DMA((_NBUF,)),
        pltpu.SemaphoreType.DMA((_NBUF,)),
        pltpu.SemaphoreType.DMA((_NBUF,)),
    ],
)


def _mlp_body(scale_ref, h_ref, agg_ref, w1_ref, b1_ref, w2_ref, b2_ref,
              out_ref):
    t = scale_ref[0] * h_ref[...] + agg_ref[0] + agg_ref[1]
    t = jnp.dot(t, w1_ref[...], preferred_element_type=jnp.float32)
    t = jnp.maximum(t + b1_ref[...], 0.0)
    t = jnp.dot(t, w2_ref[...], preferred_element_type=jnp.float32)
    out_ref[...] = jnp.maximum(t + b2_ref[...], 0.0)


_BN = 1000


def _tc_mlp(h, agg, w1, b1, w2, b2, eps_l):
    scale = (1.0 + eps_l).reshape(1)
    return pl.pallas_call(
        _mlp_body,
        grid=(_N // _BN,),
        in_specs=[
            pl.BlockSpec(memory_space=pltpu.SMEM),
            pl.BlockSpec((_BN, _D), lambda i: (i, 0)),
            pl.BlockSpec((_NC, _BN, _D), lambda i: (0, i, 0)),
            pl.BlockSpec((_D, _D), lambda i: (0, 0)),
            pl.BlockSpec((1, _D), lambda i: (0, 0)),
            pl.BlockSpec((_D, _D), lambda i: (0, 0)),
            pl.BlockSpec((1, _D), lambda i: (0, 0)),
        ],
        out_specs=pl.BlockSpec((_BN, _D), lambda i: (i, 0)),
        out_shape=jax.ShapeDtypeStruct((_N, _D), jnp.float32),
    )(scale, h, agg, w1, b1.reshape(1, _D), w2, b2.reshape(1, _D))


def kernel(x, edge_index, edge_attr, W1, b1, W2, b2, eps):
    src = edge_index[0]
    dst = edge_index[1]
    h = x
    for l in range(_L):
        agg = _sc_agg(h, src, dst, edge_attr)
        h = _tc_mlp(h, agg, W1[l], b1[l], W2[l], b2[l], eps[l])
    return h
